# baseline (speedup 1.0000x reference)
import functools

import jax
import jax.numpy as jnp
from jax.experimental import pallas as pl
from jax.experimental.pallas import tpu as pltpu

_BF16 = jnp.bfloat16
_LOG2E = 1.4426950408889634


def _call_a_body(a_ref, x0t_ref, zib_ref, zjb_ref, negb_ref,
                 x1t_ref, ell_ref, dacc_ref, *, nb, bi, bj, n):
    i = pl.program_id(0)
    r = pl.program_id(1)
    s = i * nb + r
    nsteps = pl.num_programs(0) * nb

    x1t_ref[0] = jnp.dot(a_ref[0].astype(_BF16), x0t_ref[...],
                         preferred_element_type=jnp.float32).astype(_BF16)

    @pl.when(s == 0)
    def _():
        dacc_ref[...] = jnp.zeros_like(dacc_ref)

    tiles_total = (n // bi) * (n // bj)
    tpstep = tiles_total // nsteps
    for u in range(tpstep):
        t = s * tpstep + u
        ib = t // (n // bj)
        jb = t % (n // bj)
        zit = zib_ref[pl.ds(ib * bi, bi), :]
        zjt = zjb_ref[pl.ds(jb * bj, bj), :]
        tile = jax.lax.dot_general(
            zit, zjt, (((1,), (1,)), ((), ())),
            preferred_element_type=jnp.float32)
        nbj = negb_ref[:, pl.ds(jb * bj, bj)]
        t2 = jnp.maximum(tile, nbj)
        dacc_ref[:, pl.ds(jb * bj, bj)] += jnp.sum(
            jnp.exp2(t2), axis=0, keepdims=True)

    @pl.when(s == nsteps - 1)
    def _():
        ell_ref[...] = jnp.log2(dacc_ref[...])


def _call_b_body(a_ref, x1t_ref, x0t_ref, x0t16_ref, zib_ref, zjb_ref,
                 ell_ref, negb_ref, w_ref, out_ref, ps_ref, xz_ref,
                 *, bm, bj, n, nsup, batch, d):
    r = pl.program_id(0)
    i = pl.program_id(1)
    bd = batch * d
    nsup_i = int(nsup)

    x1t = x1t_ref[0]
    rows = x1t_ref[0, pl.ds(r * bm, bm), :].astype(jnp.float32)
    acc = rows + 2.0 * jnp.dot(a_ref[0].astype(_BF16), x1t,
                               preferred_element_type=jnp.float32)

    zit = zib_ref[...]
    chunk = (n // bj) // nsup_i

    def body(k, xz):
        zjt = zjb_ref[pl.ds(k * bj, bj), :]
        t = jax.lax.dot_general(
            zit, zjt, (((1,), (1,)), ((), ())),
            preferred_element_type=jnp.float32)
        lj = ell_ref[:, pl.ds(k * bj, bj)]
        fj = negb_ref[:, pl.ds(k * bj, bj)] - lj
        t2 = jnp.maximum(t - lj, fj)
        e = jnp.exp2(t2).astype(_BF16)
        v = x0t16_ref[pl.ds(k * bj, bj), :]
        return xz + jnp.dot(e, v, preferred_element_type=jnp.float32)

    xz_part = jax.lax.fori_loop(i * chunk, (i + 1) * chunk, body,
                                jnp.zeros((bm, bd), jnp.float32))

    @pl.when(i == 0)
    def _():
        ps_ref[...] = acc + (1.0 - nsup) * x0t_ref[...]
        xz_ref[...] = xz_part

    @pl.when(i != 0)
    def _():
        ps_ref[...] += acc
        xz_ref[...] += xz_part

    @pl.when(i == nsup_i - 1)
    def _():
        s = ps_ref[...] + xz_ref[...]
        w = w_ref[...]
        for b in range(batch):
            out_ref[b] = jnp.dot(s[:, b * d:(b + 1) * d], w,
                                 preferred_element_type=jnp.float32)


def kernel(A, X, Z, W):
    nsup, n, _ = A.shape
    batch, d, _ = X.shape
    zdim = Z.shape[1]
    bd = batch * d
    out_f = W.shape[1]

    X0T = X.reshape(bd, n).T
    X0T16 = X0T.astype(_BF16)

    nrm2 = jnp.sum(Z * Z, axis=1)
    bbound = jnp.sqrt(nrm2 * jnp.max(nrm2))
    nb16 = (-bbound * _LOG2E).astype(_BF16)
    pad = jnp.zeros((n, 128 - zdim - 1), _BF16)
    zib = jnp.concatenate(
        [(Z * _LOG2E).astype(_BF16), jnp.ones((n, 1), _BF16), pad], axis=1)
    zjb = jnp.concatenate(
        [Z.astype(_BF16), nb16[:, None], pad], axis=1)
    negb = nb16.astype(jnp.float32)[None, :]
    zext = zib.shape[1]

    BM = 1024
    BI = 1024
    BJ = 1024
    nb = n // BM

    x1t, ell = pl.pallas_call(
        functools.partial(_call_a_body, nb=nb, bi=BI, bj=BJ, n=n),
        grid=(nsup, nb),
        in_specs=[
            pl.BlockSpec((1, BM, n), lambda i, r: (i, r, 0)),
            pl.BlockSpec((n, bd), lambda i, r: (0, 0)),
            pl.BlockSpec((n, zext), lambda i, r: (0, 0)),
            pl.BlockSpec((n, zext), lambda i, r: (0, 0)),
            pl.BlockSpec((1, n), lambda i, r: (0, 0)),
        ],
        out_specs=[
            pl.BlockSpec((1, BM, bd), lambda i, r: (i, r, 0)),
            pl.BlockSpec((1, n), lambda i, r: (0, 0)),
        ],
        out_shape=[
            jax.ShapeDtypeStruct((nsup, n, bd), _BF16),
            jax.ShapeDtypeStruct((1, n), jnp.float32),
        ],
        scratch_shapes=[pltpu.VMEM((1, n), jnp.float32)],
        compiler_params=pltpu.CompilerParams(
            dimension_semantics=("arbitrary", "arbitrary")),
    )(A, X0T16, zib, zjb, negb)

    out = pl.pallas_call(
        functools.partial(_call_b_body, bm=BM, bj=BJ, n=n,
                          nsup=float(nsup), batch=batch, d=d),
        grid=(nb, nsup),
        in_specs=[
            pl.BlockSpec((1, BM, n), lambda r, i: (i, r, 0)),
            pl.BlockSpec((1, n, bd), lambda r, i: (i, 0, 0)),
            pl.BlockSpec((BM, bd), lambda r, i: (r, 0)),
            pl.BlockSpec((n, bd), lambda r, i: (0, 0)),
            pl.BlockSpec((BM, zext), lambda r, i: (r, 0)),
            pl.BlockSpec((n, zext), lambda r, i: (0, 0)),
            pl.BlockSpec((1, n), lambda r, i: (0, 0)),
            pl.BlockSpec((1, n), lambda r, i: (0, 0)),
            pl.BlockSpec((d, out_f), lambda r, i: (0, 0)),
        ],
        out_specs=pl.BlockSpec((batch, BM, out_f), lambda r, i: (0, r, 0)),
        out_shape=jax.ShapeDtypeStruct((batch, n, out_f), jnp.float32),
        scratch_shapes=[pltpu.VMEM((BM, bd), jnp.float32),
                        pltpu.VMEM((BM, bd), jnp.float32)],
        compiler_params=pltpu.CompilerParams(
            dimension_semantics=("arbitrary", "arbitrary")),
    )(A, x1t, X0T, X0T16, zib, zjb, ell, negb, W)

    return out


# speedup vs baseline: 1.0185x; 1.0185x over previous
import functools

import jax
import jax.numpy as jnp
from jax.experimental import pallas as pl
from jax.experimental.pallas import tpu as pltpu

_BF16 = jnp.bfloat16
_LOG2E = 1.4426950408889634


def _fused_body(a_ref, x0t_ref, x0t16_ref, zib_ref, zjb_ref, negb_ref,
                w_ref, out_ref, x1t_ref, ps_ref, xz_ref, dacc_ref, ell_ref,
                *, bm, bi, bj, n, nsup, batch, d):
    ph = pl.program_id(0)
    i = pl.program_id(1)
    r = pl.program_id(2)
    nb = pl.num_programs(2)
    s = i * nb + r
    nsteps = nsup * nb
    bd = batch * d
    a16 = a_ref[0].astype(_BF16)

    @pl.when(ph == 0)
    def _phase0():
        x1 = jnp.dot(a16, x0t16_ref[...],
                     preferred_element_type=jnp.float32)
        x1t_ref[i, pl.ds(r * bm, bm), :] = x1.astype(_BF16)

        @pl.when(s == 0)
        def _():
            dacc_ref[...] = jnp.zeros_like(dacc_ref)

        tiles_total = (n // bi) * (n // bj)
        tpstep = tiles_total // nsteps
        for u in range(tpstep):
            t = s * tpstep + u
            ib = t // (n // bj)
            jb = t % (n // bj)
            zit = zib_ref[pl.ds(ib * bi, bi), :]
            zjt = zjb_ref[pl.ds(jb * bj, bj), :]
            tile = jax.lax.dot_general(
                zit, zjt, (((1,), (1,)), ((), ())),
                preferred_element_type=jnp.float32)
            nbj = negb_ref[:, pl.ds(jb * bj, bj)]
            t2 = jnp.maximum(tile, nbj)
            dacc_ref[:, pl.ds(jb * bj, bj)] += jnp.sum(
                jnp.exp2(t2), axis=0, keepdims=True)

        @pl.when(s == nsteps - 1)
        def _():
            ell_ref[...] = jnp.log2(dacc_ref[...])

    @pl.when(ph == 1)
    def _phase1():
        x1t = x1t_ref[i]
        rows = x1t_ref[i, pl.ds(r * bm, bm), :].astype(jnp.float32)
        acc = rows + 2.0 * jnp.dot(a16, x1t,
                                   preferred_element_type=jnp.float32)

        zit = zib_ref[pl.ds(r * bm, bm), :]
        chunk = (n // bj) // nsup

        def body(k, xz):
            zjt = zjb_ref[pl.ds(k * bj, bj), :]
            t = jax.lax.dot_general(
                zit, zjt, (((1,), (1,)), ((), ())),
                preferred_element_type=jnp.float32)
            lj = ell_ref[:, pl.ds(k * bj, bj)]
            fj = negb_ref[:, pl.ds(k * bj, bj)] - lj
            t2 = jnp.maximum(t - lj, fj)
            e = jnp.exp2(t2).astype(_BF16)
            v = x0t16_ref[pl.ds(k * bj, bj), :]
            return xz + jnp.dot(e, v, preferred_element_type=jnp.float32)

        xz_part = jax.lax.fori_loop(i * chunk, (i + 1) * chunk, body,
                                    jnp.zeros((bm, bd), jnp.float32))

        @pl.when(i == 0)
        def _():
            ps_ref[pl.ds(r * bm, bm), :] = (
                acc + (1.0 - nsup) * x0t_ref[pl.ds(r * bm, bm), :])
            xz_ref[...] = xz_part

        @pl.when(i != 0)
        def _():
            ps_ref[pl.ds(r * bm, bm), :] += acc
            xz_ref[...] += xz_part

        @pl.when(i == nsup - 1)
        def _():
            st = ps_ref[pl.ds(r * bm, bm), :] + xz_ref[...]
            w = w_ref[...]
            for b in range(batch):
                out_ref[b, pl.ds(r * bm, bm), :] = jnp.dot(
                    st[:, b * d:(b + 1) * d], w,
                    preferred_element_type=jnp.float32)


def kernel(A, X, Z, W):
    nsup, n, _ = A.shape
    batch, d, _ = X.shape
    zdim = Z.shape[1]
    bd = batch * d
    out_f = W.shape[1]

    X0T = X.reshape(bd, n).T
    X0T16 = X0T.astype(_BF16)

    nrm2 = jnp.sum(Z * Z, axis=1)
    bbound = jnp.sqrt(nrm2 * jnp.max(nrm2))
    nb16 = (-bbound * _LOG2E).astype(_BF16)
    pad = jnp.zeros((n, 128 - zdim - 1), _BF16)
    zib = jnp.concatenate(
        [(Z * _LOG2E).astype(_BF16), jnp.ones((n, 1), _BF16), pad], axis=1)
    zjb = jnp.concatenate(
        [Z.astype(_BF16), nb16[:, None], pad], axis=1)
    negb = nb16.astype(jnp.float32)[None, :]
    zext = zib.shape[1]

    BM = 1024
    BI = 1024
    BJ = 1024
    nb = n // BM

    out = pl.pallas_call(
        functools.partial(_fused_body, bm=BM, bi=BI, bj=BJ, n=n,
                          nsup=nsup, batch=batch, d=d),
        grid=(2, nsup, nb),
        in_specs=[
            pl.BlockSpec((1, BM, n), lambda ph, i, r: (i, r, 0)),
            pl.BlockSpec((n, bd), lambda ph, i, r: (0, 0)),
            pl.BlockSpec((n, bd), lambda ph, i, r: (0, 0)),
            pl.BlockSpec((n, zext), lambda ph, i, r: (0, 0)),
            pl.BlockSpec((n, zext), lambda ph, i, r: (0, 0)),
            pl.BlockSpec((1, n), lambda ph, i, r: (0, 0)),
            pl.BlockSpec((d, out_f), lambda ph, i, r: (0, 0)),
        ],
        out_specs=pl.BlockSpec((batch, n, out_f), lambda ph, i, r: (0, 0, 0)),
        out_shape=jax.ShapeDtypeStruct((batch, n, out_f), jnp.float32),
        scratch_shapes=[
            pltpu.VMEM((nsup, n, bd), _BF16),
            pltpu.VMEM((n, bd), jnp.float32),
            pltpu.VMEM((BM, bd), jnp.float32),
            pltpu.VMEM((1, n), jnp.float32),
            pltpu.VMEM((1, n), jnp.float32),
        ],
        compiler_params=pltpu.CompilerParams(
            dimension_semantics=("arbitrary", "arbitrary", "arbitrary")),
    )(A, X0T, X0T16, zib, zjb, negb, W)

    return out


# speedup vs baseline: 1.1612x; 1.1401x over previous
import functools

import jax
import jax.numpy as jnp
from jax.experimental import pallas as pl
from jax.experimental.pallas import tpu as pltpu

_BF16 = jnp.bfloat16
_LOG2E = 1.4426950408889634


def _fused_body(a_ref, x0t_ref, zib_ref, zjb_ref, negb_ref,
                w_ref, out_ref, x1t_ref, ps_ref, xz_ref, dacc_ref, ell_ref,
                *, bm, bi, bj, n, nsup, batch, d):
    ph = pl.program_id(0)
    i = pl.program_id(1)
    r = pl.program_id(2)
    nb = pl.num_programs(2)
    s = i * nb + r
    nsteps = nsup * nb
    bd = batch * d

    @pl.when(ph == 0)
    def _phase0():
        x1t_ref[i, pl.ds(r * bm, bm), :] = jnp.dot(
            a_ref[0], x0t_ref[...], preferred_element_type=jnp.float32)

        @pl.when(s == 0)
        def _():
            dacc_ref[...] = jnp.zeros_like(dacc_ref)

        tiles_total = (n // bi) * (n // bj)
        tpstep = tiles_total // nsteps
        for u in range(tpstep):
            t = s * tpstep + u
            ib = t // (n // bj)
            jb = t % (n // bj)
            zit = zib_ref[pl.ds(ib * bi, bi), :]
            zjt = zjb_ref[pl.ds(jb * bj, bj), :]
            tile = jax.lax.dot_general(
                zit, zjt, (((1,), (1,)), ((), ())),
                preferred_element_type=jnp.float32)
            nbj = negb_ref[:, pl.ds(jb * bj, bj)]
            t2 = jnp.maximum(tile, nbj)
            dacc_ref[:, pl.ds(jb * bj, bj)] += jnp.sum(
                jnp.exp2(t2), axis=0, keepdims=True)

        @pl.when(s == nsteps - 1)
        def _():
            ell_ref[...] = jnp.log2(dacc_ref[...])

    @pl.when(ph == 1)
    def _phase1():
        x1t = x1t_ref[i]
        rows = x1t_ref[i, pl.ds(r * bm, bm), :]
        acc = rows + 2.0 * jnp.dot(a_ref[0], x1t,
                                   preferred_element_type=jnp.float32)

        zit = zib_ref[pl.ds(r * bm, bm), :]
        chunk = (n // bj) // nsup

        def body(k, xz):
            zjt = zjb_ref[pl.ds(k * bj, bj), :]
            t = jax.lax.dot_general(
                zit, zjt, (((1,), (1,)), ((), ())),
                preferred_element_type=jnp.float32)
            lj = ell_ref[:, pl.ds(k * bj, bj)]
            fj = negb_ref[:, pl.ds(k * bj, bj)] - lj
            t2 = jnp.maximum(t - lj, fj)
            e = jnp.exp2(t2)
            v = x0t_ref[pl.ds(k * bj, bj), :]
            return xz + jnp.dot(e, v, preferred_element_type=jnp.float32)

        xz_part = jax.lax.fori_loop(i * chunk, (i + 1) * chunk, body,
                                    jnp.zeros((bm, bd), jnp.float32))

        @pl.when(i == 0)
        def _():
            ps_ref[pl.ds(r * bm, bm), :] = (
                acc + (1.0 - nsup) * x0t_ref[pl.ds(r * bm, bm), :])
            xz_ref[...] = xz_part

        @pl.when(i != 0)
        def _():
            ps_ref[pl.ds(r * bm, bm), :] += acc
            xz_ref[...] += xz_part

        @pl.when(i == nsup - 1)
        def _():
            st = ps_ref[pl.ds(r * bm, bm), :] + xz_ref[...]
            w = w_ref[...]
            for b in range(batch):
                out_ref[b, pl.ds(r * bm, bm), :] = jnp.dot(
                    st[:, b * d:(b + 1) * d], w,
                    preferred_element_type=jnp.float32)


def kernel(A, X, Z, W):
    nsup, n, _ = A.shape
    batch, d, _ = X.shape
    zdim = Z.shape[1]
    bd = batch * d
    out_f = W.shape[1]

    X0T = X.reshape(bd, n).T

    nrm2 = jnp.sum(Z * Z, axis=1)
    bbound = jnp.sqrt(nrm2 * jnp.max(nrm2))
    nb16 = (-bbound * _LOG2E).astype(_BF16)
    pad = jnp.zeros((n, 128 - zdim - 1), _BF16)
    zib = jnp.concatenate(
        [(Z * _LOG2E).astype(_BF16), jnp.ones((n, 1), _BF16), pad], axis=1)
    zjb = jnp.concatenate(
        [Z.astype(_BF16), nb16[:, None], pad], axis=1)
    negb = nb16.astype(jnp.float32)[None, :]
    zext = zib.shape[1]

    BM = 1024
    BI = 1024
    BJ = 1024
    nb = n // BM

    out = pl.pallas_call(
        functools.partial(_fused_body, bm=BM, bi=BI, bj=BJ, n=n,
                          nsup=nsup, batch=batch, d=d),
        grid=(2, nsup, nb),
        in_specs=[
            pl.BlockSpec((1, BM, n), lambda ph, i, r: (i, r, 0)),
            pl.BlockSpec((n, bd), lambda ph, i, r: (0, 0)),
            pl.BlockSpec((n, zext), lambda ph, i, r: (0, 0)),
            pl.BlockSpec((n, zext), lambda ph, i, r: (0, 0)),
            pl.BlockSpec((1, n), lambda ph, i, r: (0, 0)),
            pl.BlockSpec((d, out_f), lambda ph, i, r: (0, 0)),
        ],
        out_specs=pl.BlockSpec((batch, n, out_f), lambda ph, i, r: (0, 0, 0)),
        out_shape=jax.ShapeDtypeStruct((batch, n, out_f), jnp.float32),
        scratch_shapes=[
            pltpu.VMEM((nsup, n, bd), jnp.float32),
            pltpu.VMEM((n, bd), jnp.float32),
            pltpu.VMEM((BM, bd), jnp.float32),
            pltpu.VMEM((1, n), jnp.float32),
            pltpu.VMEM((1, n), jnp.float32),
        ],
        compiler_params=pltpu.CompilerParams(
            dimension_semantics=("arbitrary", "arbitrary", "arbitrary")),
    )(A, X0T, zib, zjb, negb, W)

    return out
